# Initial kernel scaffold; baseline (speedup 1.0000x reference)
import jax
import jax.numpy as jnp
from jax.experimental import pallas as pl


def kernel(x, edge_index, Wl1, bl1, Wr1, Wl2, bl2, Wr2, Wl3, bl3, Wr3, Wl4, bl4, Wr4, g1, beta1, g2, beta2, g3, beta3):
    raise NotImplementedError("write your pallas kernel here")


# speedup vs baseline: 14.9221x; 14.9221x over previous
import functools

import jax
import jax.numpy as jnp
from jax import lax
from jax.experimental import pallas as pl
from jax.experimental.pallas import tpu as pltpu
from jax.experimental.pallas import tpu_sc as plsc

_N = 100000
_E = 3200000
_NC = 2
_NS = 16
_NW = _NC * _NS
_D = 16
_CH = 80
_NCH = 125
_EPW = _E // _NW
_NSB = _EPW // (_NCH * _CH)
_RPT = _N // _NS
_ZR = 250
_B = 5000
_GRID = _N // _B


def _sc_scatter(h16, src2, dst2):
  mesh = plsc.VectorSubcoreMesh(core_axis_name="c", subcore_axis_name="s")
  scratch = [
      pltpu.VMEM((_NCH, _CH), jnp.int32),
      pltpu.VMEM((_NCH, _CH), jnp.int32),
      pltpu.VMEM((_CH, _D), jnp.float32),
      pltpu.VMEM((_ZR, _D), jnp.float32),
      pltpu.VMEM_SHARED((_N, _D), jnp.float32),
      pltpu.SemaphoreType.DMA,
  ]

  def body(h_hbm, src_hbm, dst_hbm, zd_hbm, out_hbm,
           sidx, didx, rows, zbuf, acc, sem):
    c = lax.axis_index("c")
    s = lax.axis_index("s")
    w = s * _NC + c

    pltpu.sync_copy(zd_hbm, zbuf)
    for r in range(_RPT // _ZR):
      pltpu.sync_copy(zbuf, acc.at[pl.ds(s * _RPT + r * _ZR, _ZR)])
    plsc.subcore_barrier()

    def sb_body(b, carry):
      q = w * _NSB + b
      pltpu.sync_copy(src_hbm.at[q], sidx)
      pltpu.sync_copy(dst_hbm.at[q], didx)

      def ch_body(j, carry2):
        pltpu.async_copy(h_hbm.at[sidx.at[j]], rows, sem).wait()
        pltpu.sync_copy(rows, acc.at[didx.at[j]], add=True)
        return carry2

      return lax.fori_loop(0, _NCH, ch_body, carry)

    lax.fori_loop(0, _NSB, sb_body, 0)
    plsc.subcore_barrier()

    pltpu.sync_copy(acc.at[pl.ds(s * _RPT, _RPT)],
                    out_hbm.at[c, pl.ds(s * _RPT, _RPT)])

  kfn = pl.kernel(body, out_type=jax.ShapeDtypeStruct((_NC, _N, _D), jnp.float32),
                  mesh=mesh, scratch_types=scratch,
                  compiler_params=pltpu.CompilerParams(
                      use_tc_tiling_on_sc=False))
  return kfn(h16, src2, dst2, jnp.zeros((_ZR, _D), jnp.float32))


def _row_spec(d):
  return pl.BlockSpec((_B, d), lambda i: (i, 0))


def _full_spec(shape):
  nd = len(shape)
  return pl.BlockSpec(shape, lambda i: (0,) * nd)


def _tc_layer1(x16, p0, p1, WlT, bl, WrT):
  din, dout = WlT.shape

  def body(x_ref, p0_ref, p1_ref, wl_ref, bl_ref, wr_ref,
           y_ref, inv_ref, ind_ref, st_ref):
    p = p0_ref[...] + p1_ref[...]
    cnt = p[:, din:din + 1]
    inv = 1.0 / jnp.maximum(cnt, 1.0)
    ind = cnt * inv
    mean = p[:, :din] * inv
    z = (jnp.dot(mean, wl_ref[...], preferred_element_type=jnp.float32)
         + bl_ref[...]
         + jnp.dot(x_ref[:, :din], wr_ref[...],
                   preferred_element_type=jnp.float32))
    nrm = jnp.maximum(jnp.sqrt(jnp.sum(z * z, axis=-1, keepdims=True)), 1e-12)
    y = jnp.maximum(z / nrm, 0.0)
    y_ref[...] = jnp.concatenate(
        [y, jnp.zeros((y.shape[0], _D - dout), jnp.float32)], axis=1)
    inv_ref[...] = inv
    ind_ref[...] = ind
    i = pl.program_id(0)

    @pl.when(i == 0)
    def _():
      st_ref[...] = jnp.zeros_like(st_ref)

    st_ref[...] += jnp.stack([jnp.sum(y, axis=0), jnp.sum(y * y, axis=0)])

  return pl.pallas_call(
      body,
      grid=(_GRID,),
      in_specs=[_row_spec(_D), _row_spec(_D), _row_spec(_D),
                _full_spec((din, dout)), _full_spec((dout,)),
                _full_spec((din, dout))],
      out_specs=[_row_spec(_D), _row_spec(1), _row_spec(1),
                 pl.BlockSpec((2, dout), lambda i: (0, 0))],
      out_shape=[jax.ShapeDtypeStruct((_N, _D), jnp.float32),
                 jax.ShapeDtypeStruct((_N, 1), jnp.float32),
                 jax.ShapeDtypeStruct((_N, 1), jnp.float32),
                 jax.ShapeDtypeStruct((2, dout), jnp.float32)],
  )(x16, p0, p1, WlT, bl, WrT)


def _tc_mid(h16, p0, p1, inv, ind, st, g, beta, WlT, bl, WrT, want_stats):
  din, dout = WlT.shape
  dpad = _D if dout <= _D else dout

  def body(*refs):
    if want_stats:
      (h_ref, p0_ref, p1_ref, inv_ref, ind_ref, st_ref, g_ref, b_ref,
       wl_ref, bl_ref, wr_ref, y_ref, sto_ref) = refs
    else:
      (h_ref, p0_ref, p1_ref, inv_ref, ind_ref, st_ref, g_ref, b_ref,
       wl_ref, bl_ref, wr_ref, y_ref) = refs
    m = st_ref[0, :] / _N
    v = st_ref[1, :] / _N - m * m
    sc = g_ref[...] * lax.rsqrt(v + 1e-5)
    t = b_ref[...] - m * sc
    hbn = h_ref[:, :din] * sc + t
    p = p0_ref[...] + p1_ref[...]
    meanbn = p[:, :din] * inv_ref[...] * sc + t * ind_ref[...]
    z = (jnp.dot(meanbn, wl_ref[...], preferred_element_type=jnp.float32)
         + bl_ref[...]
         + jnp.dot(hbn, wr_ref[...], preferred_element_type=jnp.float32))
    nrm = jnp.maximum(jnp.sqrt(jnp.sum(z * z, axis=-1, keepdims=True)), 1e-12)
    y = jnp.maximum(z / nrm, 0.0)
    if dpad > dout:
      y_ref[...] = jnp.concatenate(
          [y, jnp.zeros((y.shape[0], dpad - dout), jnp.float32)], axis=1)
    else:
      y_ref[...] = y
    if want_stats:
      i = pl.program_id(0)

      @pl.when(i == 0)
      def _():
        sto_ref[...] = jnp.zeros_like(sto_ref)

      sto_ref[...] += jnp.stack([jnp.sum(y, axis=0), jnp.sum(y * y, axis=0)])

  out_specs = [_row_spec(dpad)]
  out_shape = [jax.ShapeDtypeStruct((_N, dpad), jnp.float32)]
  if want_stats:
    out_specs.append(pl.BlockSpec((2, dout), lambda i: (0, 0)))
    out_shape.append(jax.ShapeDtypeStruct((2, dout), jnp.float32))
  res = pl.pallas_call(
      body,
      grid=(_GRID,),
      in_specs=[_row_spec(_D), _row_spec(_D), _row_spec(_D),
                _row_spec(1), _row_spec(1),
                _full_spec((2, din)), _full_spec((din,)), _full_spec((din,)),
                _full_spec((din, dout)), _full_spec((dout,)),
                _full_spec((din, dout))],
      out_specs=out_specs,
      out_shape=out_shape,
  )(h16, p0, p1, inv, ind, st, g, beta, WlT, bl, WrT)
  return res if want_stats else res[0]


def kernel(x, edge_index, Wl1, bl1, Wr1, Wl2, bl2, Wr2, Wl3, bl3, Wr3,
           Wl4, bl4, Wr4, g1, beta1, g2, beta2, g3, beta3):
  src2 = edge_index[0].reshape(_NW * _NSB, _NCH, _CH)
  dst2 = edge_index[1].reshape(_NW * _NSB, _NCH, _CH)
  x16 = jnp.concatenate(
      [x, jnp.ones((_N, 1), jnp.float32), jnp.zeros((_N, _D - 5), jnp.float32)],
      axis=1)

  a1 = _sc_scatter(x16, src2, dst2)
  y1, inv, ind, st1 = _tc_layer1(x16, a1[0], a1[1], Wl1.T, bl1, Wr1.T)
  a2 = _sc_scatter(y1, src2, dst2)
  y2, st2 = _tc_mid(y1, a2[0], a2[1], inv, ind, st1, g1, beta1,
                    Wl2.T, bl2, Wr2.T, True)
  a3 = _sc_scatter(y2, src2, dst2)
  y3, st3 = _tc_mid(y2, a3[0], a3[1], inv, ind, st2, g2, beta2,
                    Wl3.T, bl3, Wr3.T, True)
  a4 = _sc_scatter(y3, src2, dst2)
  y4 = _tc_mid(y3, a4[0], a4[1], inv, ind, st3, g3, beta3,
               Wl4.T, bl4, Wr4.T, False)
  return y4


# speedup vs baseline: 20.1761x; 1.3521x over previous
import functools

import jax
import jax.numpy as jnp
from jax import lax
from jax.experimental import pallas as pl
from jax.experimental.pallas import tpu as pltpu
from jax.experimental.pallas import tpu_sc as plsc

_N = 100000
_E = 3200000
_NC = 2
_NS = 16
_NW = _NC * _NS
_D = 16
_CH = 125
_NCH = 80
_EPW = _E // _NW
_NSB = _EPW // (_NCH * _CH)
_RPT = _N // _NS
_ZR = 250
_B = 5000
_GRID = _N // _B


def _sc_scatter(h16, src2, dst2):
  mesh = plsc.VectorSubcoreMesh(core_axis_name="c", subcore_axis_name="s")
  scratch = [
      pltpu.VMEM((_NCH, _CH), jnp.int32),
      pltpu.VMEM((_NCH, _CH), jnp.int32),
      pltpu.VMEM((_CH, _D), jnp.float32),
      pltpu.VMEM((_CH, _D), jnp.float32),
      pltpu.VMEM((_ZR, _D), jnp.float32),
      pltpu.VMEM_SHARED((_N, _D), jnp.float32),
      pltpu.SemaphoreType.DMA,
      pltpu.SemaphoreType.DMA,
  ]

  def body(h_hbm, src_hbm, dst_hbm, zd_hbm, out_hbm,
           sidx, didx, rows0, rows1, zbuf, acc, sem0, sem1):
    c = lax.axis_index("c")
    s = lax.axis_index("s")
    w = s * _NC + c

    pltpu.sync_copy(zd_hbm, zbuf)
    for r in range(_RPT // _ZR):
      pltpu.sync_copy(zbuf, acc.at[pl.ds(s * _RPT + r * _ZR, _ZR)])
    plsc.subcore_barrier()

    def sb_body(b, carry):
      q = w * _NSB + b
      pltpu.sync_copy(src_hbm.at[q], sidx)
      pltpu.sync_copy(dst_hbm.at[q], didx)
      pltpu.async_copy(h_hbm.at[sidx.at[0]], rows0, sem0)

      def pair_body(jj, carry2):
        j0 = jj * 2
        j1 = j0 + 1
        pltpu.make_async_copy(h_hbm.at[sidx.at[j0]], rows0, sem0).wait()
        pltpu.async_copy(h_hbm.at[sidx.at[j1]], rows1, sem1)
        pltpu.sync_copy(rows0, acc.at[didx.at[j0]], add=True)
        pltpu.make_async_copy(h_hbm.at[sidx.at[j1]], rows1, sem1).wait()

        @pl.when(j1 + 1 < _NCH)
        def _():
          pltpu.async_copy(h_hbm.at[sidx.at[j1 + 1]], rows0, sem0)

        pltpu.sync_copy(rows1, acc.at[didx.at[j1]], add=True)
        return carry2

      return lax.fori_loop(0, _NCH // 2, pair_body, carry)

    lax.fori_loop(0, _NSB, sb_body, 0)
    plsc.subcore_barrier()

    pltpu.sync_copy(acc.at[pl.ds(s * _RPT, _RPT)],
                    out_hbm.at[c, pl.ds(s * _RPT, _RPT)])

  kfn = pl.kernel(body, out_type=jax.ShapeDtypeStruct((_NC, _N, _D), jnp.float32),
                  mesh=mesh, scratch_types=scratch,
                  compiler_params=pltpu.CompilerParams(
                      use_tc_tiling_on_sc=False))
  return kfn(h16, src2, dst2, jnp.zeros((_ZR, _D), jnp.float32))


def _row_spec(d):
  return pl.BlockSpec((_B, d), lambda i: (i, 0))


def _full_spec(shape):
  nd = len(shape)
  return pl.BlockSpec(shape, lambda i: (0,) * nd)


def _tc_layer1(x16, p0, p1, WlT, bl, WrT):
  din, dout = WlT.shape

  def body(x_ref, p0_ref, p1_ref, wl_ref, bl_ref, wr_ref,
           y_ref, inv_ref, ind_ref, st_ref):
    p = p0_ref[...] + p1_ref[...]
    cnt = p[:, din:din + 1]
    inv = 1.0 / jnp.maximum(cnt, 1.0)
    ind = cnt * inv
    mean = p[:, :din] * inv
    z = (jnp.dot(mean, wl_ref[...], preferred_element_type=jnp.float32)
         + bl_ref[...]
         + jnp.dot(x_ref[:, :din], wr_ref[...],
                   preferred_element_type=jnp.float32))
    nrm = jnp.maximum(jnp.sqrt(jnp.sum(z * z, axis=-1, keepdims=True)), 1e-12)
    y = jnp.maximum(z / nrm, 0.0)
    y_ref[...] = jnp.concatenate(
        [y, jnp.zeros((y.shape[0], _D - dout), jnp.float32)], axis=1)
    inv_ref[...] = inv
    ind_ref[...] = ind
    i = pl.program_id(0)

    @pl.when(i == 0)
    def _():
      st_ref[...] = jnp.zeros_like(st_ref)

    st_ref[...] += jnp.stack([jnp.sum(y, axis=0), jnp.sum(y * y, axis=0)])

  return pl.pallas_call(
      body,
      grid=(_GRID,),
      in_specs=[_row_spec(_D), _row_spec(_D), _row_spec(_D),
                _full_spec((din, dout)), _full_spec((dout,)),
                _full_spec((din, dout))],
      out_specs=[_row_spec(_D), _row_spec(1), _row_spec(1),
                 pl.BlockSpec((2, dout), lambda i: (0, 0))],
      out_shape=[jax.ShapeDtypeStruct((_N, _D), jnp.float32),
                 jax.ShapeDtypeStruct((_N, 1), jnp.float32),
                 jax.ShapeDtypeStruct((_N, 1), jnp.float32),
                 jax.ShapeDtypeStruct((2, dout), jnp.float32)],
  )(x16, p0, p1, WlT, bl, WrT)


def _tc_mid(h16, p0, p1, inv, ind, st, g, beta, WlT, bl, WrT, want_stats):
  din, dout = WlT.shape
  dpad = _D if dout <= _D else dout

  def body(*refs):
    if want_stats:
      (h_ref, p0_ref, p1_ref, inv_ref, ind_ref, st_ref, g_ref, b_ref,
       wl_ref, bl_ref, wr_ref, y_ref, sto_ref) = refs
    else:
      (h_ref, p0_ref, p1_ref, inv_ref, ind_ref, st_ref, g_ref, b_ref,
       wl_ref, bl_ref, wr_ref, y_ref) = refs
    m = st_ref[0, :] / _N
    v = st_ref[1, :] / _N - m * m
    sc = g_ref[...] * lax.rsqrt(v + 1e-5)
    t = b_ref[...] - m * sc
    hbn = h_ref[:, :din] * sc + t
    p = p0_ref[...] + p1_ref[...]
    meanbn = p[:, :din] * inv_ref[...] * sc + t * ind_ref[...]
    z = (jnp.dot(meanbn, wl_ref[...], preferred_element_type=jnp.float32)
         + bl_ref[...]
         + jnp.dot(hbn, wr_ref[...], preferred_element_type=jnp.float32))
    nrm = jnp.maximum(jnp.sqrt(jnp.sum(z * z, axis=-1, keepdims=True)), 1e-12)
    y = jnp.maximum(z / nrm, 0.0)
    if dpad > dout:
      y_ref[...] = jnp.concatenate(
          [y, jnp.zeros((y.shape[0], dpad - dout), jnp.float32)], axis=1)
    else:
      y_ref[...] = y
    if want_stats:
      i = pl.program_id(0)

      @pl.when(i == 0)
      def _():
        sto_ref[...] = jnp.zeros_like(sto_ref)

      sto_ref[...] += jnp.stack([jnp.sum(y, axis=0), jnp.sum(y * y, axis=0)])

  out_specs = [_row_spec(dpad)]
  out_shape = [jax.ShapeDtypeStruct((_N, dpad), jnp.float32)]
  if want_stats:
    out_specs.append(pl.BlockSpec((2, dout), lambda i: (0, 0)))
    out_shape.append(jax.ShapeDtypeStruct((2, dout), jnp.float32))
  res = pl.pallas_call(
      body,
      grid=(_GRID,),
      in_specs=[_row_spec(_D), _row_spec(_D), _row_spec(_D),
                _row_spec(1), _row_spec(1),
                _full_spec((2, din)), _full_spec((din,)), _full_spec((din,)),
                _full_spec((din, dout)), _full_spec((dout,)),
                _full_spec((din, dout))],
      out_specs=out_specs,
      out_shape=out_shape,
  )(h16, p0, p1, inv, ind, st, g, beta, WlT, bl, WrT)
  return res if want_stats else res[0]


def kernel(x, edge_index, Wl1, bl1, Wr1, Wl2, bl2, Wr2, Wl3, bl3, Wr3,
           Wl4, bl4, Wr4, g1, beta1, g2, beta2, g3, beta3):
  src2 = edge_index[0].reshape(_NW * _NSB, _NCH, _CH)
  dst2 = edge_index[1].reshape(_NW * _NSB, _NCH, _CH)
  x16 = jnp.concatenate(
      [x, jnp.ones((_N, 1), jnp.float32), jnp.zeros((_N, _D - 5), jnp.float32)],
      axis=1)

  a1 = _sc_scatter(x16, src2, dst2)
  y1, inv, ind, st1 = _tc_layer1(x16, a1[0], a1[1], Wl1.T, bl1, Wr1.T)
  a2 = _sc_scatter(y1, src2, dst2)
  y2, st2 = _tc_mid(y1, a2[0], a2[1], inv, ind, st1, g1, beta1,
                    Wl2.T, bl2, Wr2.T, True)
  a3 = _sc_scatter(y2, src2, dst2)
  y3, st3 = _tc_mid(y2, a3[0], a3[1], inv, ind, st2, g2, beta2,
                    Wl3.T, bl3, Wr3.T, True)
  a4 = _sc_scatter(y3, src2, dst2)
  y4 = _tc_mid(y3, a4[0], a4[1], inv, ind, st3, g3, beta3,
               Wl4.T, bl4, Wr4.T, False)
  return y4


# speedup vs baseline: 38.9952x; 1.9327x over previous
import functools

import jax
import jax.numpy as jnp
from jax import lax
from jax.experimental import pallas as pl
from jax.experimental.pallas import tpu as pltpu
from jax.experimental.pallas import tpu_sc as plsc

_N = 100000
_E = 3200000
_NC = 2
_NS = 16
_NW = _NC * _NS
_D = 16
_CH = 125
_NCH = 80
_EPW = _E // _NW
_NSB = _EPW // (_NCH * _CH)
_RPT = _N // _NS
_ZR = 125
_B = 5000
_GRID = _N // _B


def _sc_scatter(h16, src2, dst2):
  mesh = plsc.VectorSubcoreMesh(core_axis_name="c", subcore_axis_name="s")
  scratch = [
      pltpu.VMEM((_NCH, _CH), jnp.int32),
      pltpu.VMEM((_NCH, _CH), jnp.int32),
      pltpu.VMEM((_CH, _D), jnp.float32),
      pltpu.VMEM((_CH, _D), jnp.float32),
      pltpu.VMEM((_CH, _D), jnp.float32),
      pltpu.VMEM((_CH, _D), jnp.float32),
      pltpu.VMEM((_ZR, _D), jnp.float32),
      pltpu.VMEM_SHARED((_N, _D), jnp.float32),
      pltpu.SemaphoreType.DMA,
      pltpu.SemaphoreType.DMA,
      pltpu.SemaphoreType.DMA,
      pltpu.SemaphoreType.DMA,
  ]

  def body(h_hbm, src_hbm, dst_hbm, zd_hbm, out_hbm,
           sidx, didx, rows0, rows1, rows2, rows3, zbuf, acc,
           sem0, sem1, sem2, sem3):
    rows = (rows0, rows1, rows2, rows3)
    sems = (sem0, sem1, sem2, sem3)
    c = lax.axis_index("c")
    s = lax.axis_index("s")
    w = s * _NC + c

    pltpu.sync_copy(zd_hbm, zbuf)
    for r in range(_RPT // _ZR):
      pltpu.sync_copy(zbuf, acc.at[pl.ds(s * _RPT + r * _ZR, _ZR)])
    plsc.subcore_barrier()

    def sb_body(b, carry):
      q = w * _NSB + b
      pltpu.sync_copy(src_hbm.at[q], sidx)
      pltpu.sync_copy(dst_hbm.at[q], didx)
      for t in range(4):
        pltpu.async_copy(h_hbm.at[sidx.at[t]], rows[t], sems[t])

      def quad_body(jq, carry2):
        j0 = jq * 4
        for t in range(4):
          j = j0 + t
          pltpu.make_async_copy(h_hbm.at[sidx.at[j]], rows[t], sems[t]).wait()
          pltpu.sync_copy(rows[t], acc.at[didx.at[j]], add=True)

          @pl.when(j + 4 < _NCH)
          def _():
            pltpu.async_copy(h_hbm.at[sidx.at[j + 4]], rows[t], sems[t])

        return carry2

      return lax.fori_loop(0, _NCH // 4, quad_body, carry)

    lax.fori_loop(0, _NSB, sb_body, 0)
    plsc.subcore_barrier()

    pltpu.sync_copy(acc.at[pl.ds(s * _RPT, _RPT)],
                    out_hbm.at[c, pl.ds(s * _RPT, _RPT)])

  kfn = pl.kernel(body, out_type=jax.ShapeDtypeStruct((_NC, _N, _D), jnp.float32),
                  mesh=mesh, scratch_types=scratch,
                  compiler_params=pltpu.CompilerParams(
                      use_tc_tiling_on_sc=False))
  return kfn(h16, src2, dst2, jnp.zeros((_ZR, _D), jnp.float32))


def _row_spec(d):
  return pl.BlockSpec((_B, d), lambda i: (i, 0))


def _full_spec(shape):
  nd = len(shape)
  return pl.BlockSpec(shape, lambda i: (0,) * nd)


def _tc_layer1(x16, a, WlT, bl, WrT):
  din, dout = WlT.shape

  def body(x_ref, a_ref, wl_ref, bl_ref, wr_ref,
           y_ref, inv_ref, ind_ref, st_ref):
    p = a_ref[0] + a_ref[1]
    cnt = p[:, din:din + 1]
    inv = 1.0 / jnp.maximum(cnt, 1.0)
    ind = cnt * inv
    mean = p[:, :din] * inv
    z = (jnp.dot(mean, wl_ref[...], preferred_element_type=jnp.float32)
         + bl_ref[...]
         + jnp.dot(x_ref[:, :din], wr_ref[...],
                   preferred_element_type=jnp.float32))
    nrm = jnp.maximum(jnp.sqrt(jnp.sum(z * z, axis=-1, keepdims=True)), 1e-12)
    y = jnp.maximum(z / nrm, 0.0)
    y_ref[...] = jnp.concatenate(
        [y, jnp.zeros((y.shape[0], _D - dout), jnp.float32)], axis=1)
    inv_ref[...] = inv
    ind_ref[...] = ind
    i = pl.program_id(0)

    @pl.when(i == 0)
    def _():
      st_ref[...] = jnp.zeros_like(st_ref)

    st_ref[...] += jnp.stack([jnp.sum(y, axis=0), jnp.sum(y * y, axis=0)])

  return pl.pallas_call(
      body,
      grid=(_GRID,),
      in_specs=[_row_spec(_D),
                pl.BlockSpec((2, _B, _D), lambda i: (0, i, 0)),
                _full_spec((din, dout)), _full_spec((dout,)),
                _full_spec((din, dout))],
      out_specs=[_row_spec(_D), _row_spec(1), _row_spec(1),
                 pl.BlockSpec((2, dout), lambda i: (0, 0))],
      out_shape=[jax.ShapeDtypeStruct((_N, _D), jnp.float32),
                 jax.ShapeDtypeStruct((_N, 1), jnp.float32),
                 jax.ShapeDtypeStruct((_N, 1), jnp.float32),
                 jax.ShapeDtypeStruct((2, dout), jnp.float32)],
  )(x16, a, WlT, bl, WrT)


def _tc_mid(h16, a, inv, ind, st, g, beta, WlT, bl, WrT, want_stats):
  din, dout = WlT.shape
  dpad = _D if dout <= _D else dout

  def body(*refs):
    if want_stats:
      (h_ref, a_ref, inv_ref, ind_ref, st_ref, g_ref, b_ref,
       wl_ref, bl_ref, wr_ref, y_ref, sto_ref) = refs
    else:
      (h_ref, a_ref, inv_ref, ind_ref, st_ref, g_ref, b_ref,
       wl_ref, bl_ref, wr_ref, y_ref) = refs
    m = st_ref[0, :] / _N
    v = st_ref[1, :] / _N - m * m
    sc = g_ref[...] * lax.rsqrt(v + 1e-5)
    t = b_ref[...] - m * sc
    hbn = h_ref[:, :din] * sc + t
    p = a_ref[0] + a_ref[1]
    meanbn = p[:, :din] * inv_ref[...] * sc + t * ind_ref[...]
    z = (jnp.dot(meanbn, wl_ref[...], preferred_element_type=jnp.float32)
         + bl_ref[...]
         + jnp.dot(hbn, wr_ref[...], preferred_element_type=jnp.float32))
    nrm = jnp.maximum(jnp.sqrt(jnp.sum(z * z, axis=-1, keepdims=True)), 1e-12)
    y = jnp.maximum(z / nrm, 0.0)
    if dpad > dout:
      y_ref[...] = jnp.concatenate(
          [y, jnp.zeros((y.shape[0], dpad - dout), jnp.float32)], axis=1)
    else:
      y_ref[...] = y
    if want_stats:
      i = pl.program_id(0)

      @pl.when(i == 0)
      def _():
        sto_ref[...] = jnp.zeros_like(sto_ref)

      sto_ref[...] += jnp.stack([jnp.sum(y, axis=0), jnp.sum(y * y, axis=0)])

  out_specs = [_row_spec(dpad)]
  out_shape = [jax.ShapeDtypeStruct((_N, dpad), jnp.float32)]
  if want_stats:
    out_specs.append(pl.BlockSpec((2, dout), lambda i: (0, 0)))
    out_shape.append(jax.ShapeDtypeStruct((2, dout), jnp.float32))
  res = pl.pallas_call(
      body,
      grid=(_GRID,),
      in_specs=[_row_spec(_D),
                pl.BlockSpec((2, _B, _D), lambda i: (0, i, 0)),
                _row_spec(1), _row_spec(1),
                _full_spec((2, din)), _full_spec((din,)), _full_spec((din,)),
                _full_spec((din, dout)), _full_spec((dout,)),
                _full_spec((din, dout))],
      out_specs=out_specs,
      out_shape=out_shape,
  )(h16, a, inv, ind, st, g, beta, WlT, bl, WrT)
  return res if want_stats else res[0]


def kernel(x, edge_index, Wl1, bl1, Wr1, Wl2, bl2, Wr2, Wl3, bl3, Wr3,
           Wl4, bl4, Wr4, g1, beta1, g2, beta2, g3, beta3):
  src2 = edge_index[0].reshape(_NW * _NSB, _NCH, _CH)
  dst2 = edge_index[1].reshape(_NW * _NSB, _NCH, _CH)
  x16 = jnp.concatenate(
      [x, jnp.ones((_N, 1), jnp.float32), jnp.zeros((_N, _D - 5), jnp.float32)],
      axis=1)

  a1 = _sc_scatter(x16, src2, dst2)
  y1, inv, ind, st1 = _tc_layer1(x16, a1, Wl1.T, bl1, Wr1.T)
  a2 = _sc_scatter(y1, src2, dst2)
  y2, st2 = _tc_mid(y1, a2, inv, ind, st1, g1, beta1,
                    Wl2.T, bl2, Wr2.T, True)
  a3 = _sc_scatter(y2, src2, dst2)
  y3, st3 = _tc_mid(y2, a3, inv, ind, st2, g2, beta2,
                    Wl3.T, bl3, Wr3.T, True)
  a4 = _sc_scatter(y3, src2, dst2)
  y4 = _tc_mid(y3, a4, inv, ind, st3, g3, beta3,
               Wl4.T, bl4, Wr4.T, False)
  return y4


# speedup vs baseline: 49.4083x; 1.2670x over previous
import functools

import jax
import jax.numpy as jnp
from jax import lax
from jax.experimental import pallas as pl
from jax.experimental.pallas import tpu as pltpu
from jax.experimental.pallas import tpu_sc as plsc

_N = 100000
_E = 3200000
_NC = 2
_NS = 16
_NW = _NC * _NS
_D = 16
_CH = 125
_NCH = 80
_EPW = _E // _NW
_NSB = _EPW // (_NCH * _CH)
_RPT = _N // _NS
_ZR = 125
_B = 2000
_GRID = _N // _B


def _sc_scatter(h16, src2, dst2):
  mesh = plsc.VectorSubcoreMesh(core_axis_name="c", subcore_axis_name="s")
  scratch = [
      pltpu.VMEM((_NCH, _CH), jnp.int32),
      pltpu.VMEM((_NCH, _CH), jnp.int32),
      pltpu.VMEM((_CH, _D), jnp.float32),
      pltpu.VMEM((_CH, _D), jnp.float32),
      pltpu.VMEM((_CH, _D), jnp.float32),
      pltpu.VMEM((_CH, _D), jnp.float32),
      pltpu.VMEM((_ZR, _D), jnp.float32),
      pltpu.VMEM_SHARED((_N, _D), jnp.float32),
      pltpu.SemaphoreType.DMA,
      pltpu.SemaphoreType.DMA,
      pltpu.SemaphoreType.DMA,
      pltpu.SemaphoreType.DMA,
  ]

  def body(h_hbm, src_hbm, dst_hbm, zd_hbm, out_hbm,
           sidx, didx, rows0, rows1, rows2, rows3, zbuf, acc,
           sem0, sem1, sem2, sem3):
    rows = (rows0, rows1, rows2, rows3)
    sems = (sem0, sem1, sem2, sem3)
    c = lax.axis_index("c")
    s = lax.axis_index("s")
    w = s * _NC + c

    pltpu.sync_copy(zd_hbm, zbuf)
    for r in range(_RPT // _ZR):
      pltpu.sync_copy(zbuf, acc.at[pl.ds(s * _RPT + r * _ZR, _ZR)])
    plsc.subcore_barrier()

    def sb_body(b, carry):
      q = w * _NSB + b
      pltpu.sync_copy(src_hbm.at[q], sidx)
      pltpu.sync_copy(dst_hbm.at[q], didx)
      for t in range(4):
        pltpu.async_copy(h_hbm.at[sidx.at[t]], rows[t], sems[t])

      def quad_body(jq, carry2):
        j0 = jq * 4
        for t in range(4):
          j = j0 + t
          pltpu.make_async_copy(h_hbm.at[sidx.at[j]], rows[t], sems[t]).wait()
          pltpu.sync_copy(rows[t], acc.at[didx.at[j]], add=True)

          @pl.when(j + 4 < _NCH)
          def _():
            pltpu.async_copy(h_hbm.at[sidx.at[j + 4]], rows[t], sems[t])

        return carry2

      return lax.fori_loop(0, _NCH // 4, quad_body, carry)

    lax.fori_loop(0, _NSB, sb_body, 0)
    plsc.subcore_barrier()

    pltpu.sync_copy(acc.at[pl.ds(s * _RPT, _RPT)],
                    out_hbm.at[c, pl.ds(s * _RPT, _RPT)])

  kfn = pl.kernel(body, out_type=jax.ShapeDtypeStruct((_NC, _N, _D), jnp.float32),
                  mesh=mesh, scratch_types=scratch,
                  compiler_params=pltpu.CompilerParams(
                      use_tc_tiling_on_sc=False))
  return kfn(h16, src2, dst2, jnp.zeros((_ZR, _D), jnp.float32))


def _full_spec(shape):
  nd = len(shape)
  return pl.BlockSpec(shape, lambda i: (0,) * nd)


_BR = 250
_GB = _N // 8 // _BR


def _lane_spec(w=128):
  return pl.BlockSpec((1, _BR, w), lambda i: (i, 0, 0))


def _kron8(w16):
  return jnp.kron(jnp.eye(8, dtype=jnp.float32), w16)


def _pad16(m, din, dout, dpad=16):
  out = jnp.zeros((16, dpad), jnp.float32)
  return out.at[:din, :dout].set(m)


def _tile128(v, dout, dpad=16):
  return jnp.tile(jnp.zeros((dpad,), jnp.float32).at[:dout].set(v), 8)


def _tc_layer1(xl, al, bdl, bl128, bdr, cntbd, onesbd):
  def body(x_ref, a_ref, wl_ref, bl_ref, wr_ref, cb_ref, ob_ref,
           y_ref, inv_ref, ind_ref, st_ref):
    p = a_ref[0, 0] + a_ref[1, 0]
    cnt = jnp.dot(p, cb_ref[...], preferred_element_type=jnp.float32, precision=lax.Precision.HIGHEST)
    inv = 1.0 / jnp.maximum(cnt, 1.0)
    ind = cnt * inv
    z = (jnp.dot(p * inv, wl_ref[...], preferred_element_type=jnp.float32, precision=lax.Precision.HIGHEST)
         + bl_ref[...]
         + jnp.dot(x_ref[0], wr_ref[...], preferred_element_type=jnp.float32, precision=lax.Precision.HIGHEST))
    nrm = jnp.maximum(jnp.sqrt(jnp.dot(
        z * z, ob_ref[...], preferred_element_type=jnp.float32, precision=lax.Precision.HIGHEST)), 1e-12)
    y = jnp.maximum(z / nrm, 0.0)
    y_ref[0] = y
    inv_ref[0] = inv
    ind_ref[0] = ind
    i = pl.program_id(0)

    @pl.when(i == 0)
    def _():
      st_ref[...] = jnp.zeros_like(st_ref)

    st_ref[...] += jnp.stack([jnp.sum(y, axis=0), jnp.sum(y * y, axis=0)])

  return pl.pallas_call(
      body,
      grid=(_GB,),
      in_specs=[_lane_spec(),
                pl.BlockSpec((2, 1, _BR, 128), lambda i: (0, i, 0, 0)),
                _full_spec((128, 128)), _full_spec((128,)),
                _full_spec((128, 128)), _full_spec((128, 128)),
                _full_spec((128, 128))],
      out_specs=[_lane_spec(), _lane_spec(), _lane_spec(),
                 pl.BlockSpec((2, 128), lambda i: (0, 0))],
      out_shape=[jax.ShapeDtypeStruct((_GB, _BR, 128), jnp.float32),
                 jax.ShapeDtypeStruct((_GB, _BR, 128), jnp.float32),
                 jax.ShapeDtypeStruct((_GB, _BR, 128), jnp.float32),
                 jax.ShapeDtypeStruct((2, 128), jnp.float32)],
  )(xl, al, bdl, bl128, bdr, cntbd, onesbd)


def _tc_mid(hl, al, invl, indl, st, g128, beta128, k8, bdl, bl_t, bdr,
            onesbd, dw, want_stats):
  def body(*refs):
    if want_stats:
      (h_ref, a_ref, inv_ref, ind_ref, st_ref, g_ref, b_ref, k8_ref,
       wl_ref, bl_ref, wr_ref, ob_ref, y_ref, sto_ref) = refs
    else:
      (h_ref, a_ref, inv_ref, ind_ref, st_ref, g_ref, b_ref, k8_ref,
       wl_ref, bl_ref, wr_ref, ob_ref, y_ref) = refs
    m = jnp.dot(st_ref[0:1, :], k8_ref[...],
                preferred_element_type=jnp.float32, precision=lax.Precision.HIGHEST) / _N
    v = jnp.dot(st_ref[1:2, :], k8_ref[...],
                preferred_element_type=jnp.float32, precision=lax.Precision.HIGHEST) / _N - m * m
    sc = g_ref[...] * lax.rsqrt(v + 1e-5)
    t = b_ref[...] - m * sc
    hbn = h_ref[0] * sc + t
    p = a_ref[0, 0] + a_ref[1, 0]
    meanbn = p * inv_ref[0] * sc + t * ind_ref[0]
    z = (jnp.dot(meanbn, wl_ref[...], preferred_element_type=jnp.float32, precision=lax.Precision.HIGHEST)
         + bl_ref[...]
         + jnp.dot(hbn, wr_ref[...], preferred_element_type=jnp.float32, precision=lax.Precision.HIGHEST))
    nrm = jnp.maximum(jnp.sqrt(jnp.dot(
        z * z, ob_ref[...], preferred_element_type=jnp.float32, precision=lax.Precision.HIGHEST)), 1e-12)
    y = jnp.maximum(z / nrm, 0.0)
    y_ref[0] = y
    if want_stats:
      i = pl.program_id(0)

      @pl.when(i == 0)
      def _():
        sto_ref[...] = jnp.zeros_like(sto_ref)

      sto_ref[...] += jnp.stack([jnp.sum(y, axis=0), jnp.sum(y * y, axis=0)])

  out_specs = [_lane_spec(dw)]
  out_shape = [jax.ShapeDtypeStruct((_GB, _BR, dw), jnp.float32)]
  if want_stats:
    out_specs.append(pl.BlockSpec((2, 128), lambda i: (0, 0)))
    out_shape.append(jax.ShapeDtypeStruct((2, 128), jnp.float32))
  res = pl.pallas_call(
      body,
      grid=(_GB,),
      in_specs=[_lane_spec(),
                pl.BlockSpec((2, 1, _BR, 128), lambda i: (0, i, 0, 0)),
                _lane_spec(), _lane_spec(),
                _full_spec((2, 128)), _full_spec((128,)), _full_spec((128,)),
                _full_spec((128, 128)),
                _full_spec((128, dw)), _full_spec((dw,)),
                _full_spec((128, dw)), _full_spec((dw, dw))],
      out_specs=out_specs,
      out_shape=out_shape,
  )(hl, al, invl, indl, st, g128, beta128, k8, bdl, bl_t, bdr, onesbd)
  return res if want_stats else res[0]


def kernel(x, edge_index, Wl1, bl1, Wr1, Wl2, bl2, Wr2, Wl3, bl3, Wr3,
           Wl4, bl4, Wr4, g1, beta1, g2, beta2, g3, beta3):
  src2 = edge_index[0].reshape(_NW * _NSB, _NCH, _CH)
  dst2 = edge_index[1].reshape(_NW * _NSB, _NCH, _CH)
  x16 = jnp.concatenate(
      [x, jnp.ones((_N, 1), jnp.float32), jnp.zeros((_N, _D - 5), jnp.float32)],
      axis=1)
  xl = x16.reshape(_GB, _BR, 128)

  onesbd16 = _kron8(jnp.ones((16, 16), jnp.float32))
  onesbd32 = _kron8(jnp.ones((32, 32), jnp.float32))
  k8 = jnp.kron(jnp.ones((8, 8), jnp.float32), jnp.eye(16, dtype=jnp.float32))
  cntbd = _kron8(jnp.zeros((16, 16), jnp.float32).at[4, :].set(1.0))

  a1 = _sc_scatter(x16, src2, dst2)
  yl1, invl, indl, st1 = _tc_layer1(
      xl, a1.reshape(2, _GB, _BR, 128),
      _kron8(_pad16(Wl1.T, 4, 6)), _tile128(bl1, 6),
      _kron8(_pad16(Wr1.T, 4, 6)), cntbd, onesbd16)
  a2 = _sc_scatter(yl1.reshape(_N, _D), src2, dst2)
  yl2, st2 = _tc_mid(yl1, a2.reshape(2, _GB, _BR, 128), invl, indl, st1,
                     _tile128(g1, 6), _tile128(beta1, 6), k8,
                     _kron8(_pad16(Wl2.T, 6, 8)), _tile128(bl2, 8),
                     _kron8(_pad16(Wr2.T, 6, 8)), onesbd16, 128, True)
  a3 = _sc_scatter(yl2.reshape(_N, _D), src2, dst2)
  yl3, st3 = _tc_mid(yl2, a3.reshape(2, _GB, _BR, 128), invl, indl, st2,
                     _tile128(g2, 8), _tile128(beta2, 8), k8,
                     _kron8(_pad16(Wl3.T, 8, 16)), _tile128(bl3, 16),
                     _kron8(_pad16(Wr3.T, 8, 16)), onesbd16, 128, True)
  a4 = _sc_scatter(yl3.reshape(_N, _D), src2, dst2)
  yl4 = _tc_mid(yl3, a4.reshape(2, _GB, _BR, 128), invl, indl, st3,
                _tile128(g3, 16), _tile128(beta3, 16), k8,
                _kron8(_pad16(Wl4.T, 16, 32, 32)), _tile128(bl4, 32, 32),
                _kron8(_pad16(Wr4.T, 16, 32, 32)), onesbd32, 256, False)
  return yl4.reshape(_N, 32)
